# baseline (speedup 1.0000x reference)
import functools

import jax
import jax.numpy as jnp
from jax import lax
from jax.experimental import pallas as pl
from jax.experimental.pallas import tpu as pltpu
from jax.experimental.pallas import tpu_sc as plsc

_SIGMA = 0.5
_THRESH = 0.05
_N = 5000
_NP = 5120
_TILE = 512
_NB = _NP // _TILE
_RT = 256
_ROWS = 64
_COLS = 80
_WROWS = 2


def _rank_body(s_col_ref, s_row_ref, rank_ref):
    i = pl.program_id(0)
    s_i = s_col_ref[...]
    s_j = s_row_ref[...]
    gi = i * _RT + lax.broadcasted_iota(jnp.int32, (_RT, _NP), 0)
    gj = lax.broadcasted_iota(jnp.int32, (_RT, _NP), 1)
    before = (s_j > s_i) | ((s_j == s_i) & (gj < gi))
    rank_ref[...] = jnp.sum(before.astype(jnp.int32), axis=1, keepdims=True)


def _compute_rank(s_pad):
    return pl.pallas_call(
        _rank_body,
        grid=(_NP // _RT,),
        in_specs=[
            pl.BlockSpec((_RT, 1), lambda i: (i, 0)),
            pl.BlockSpec((1, _NP), lambda i: (0, 0)),
        ],
        out_specs=pl.BlockSpec((_RT, 1), lambda i: (i, 0)),
        out_shape=jax.ShapeDtypeStruct((_NP, 1), jnp.int32),
    )(s_pad.reshape(_NP, 1), s_pad.reshape(1, _NP))


def _sc_permute(rank2d, x1, y1, x2, y2, s):
    mesh = plsc.VectorSubcoreMesh(core_axis_name="c", subcore_axis_name="s")

    @functools.partial(
        pl.kernel,
        mesh=mesh,
        out_type=[jax.ShapeDtypeStruct((_NP,), jnp.float32)] * 5,
        scratch_types=[
            pltpu.VMEM((_WROWS, _COLS), jnp.int32),
            pltpu.VMEM((_WROWS, _COLS), jnp.float32),
            pltpu.SemaphoreType.DMA,
        ],
    )
    def permute(rank_h, x1_h, y1_h, x2_h, y2_h, s_h,
                ox1, oy1, ox2, oy2, os_, idx_v, val_v, sem):
        w = lax.axis_index("s") * 2 + lax.axis_index("c")
        base = w * _WROWS
        pltpu.sync_copy(rank_h.at[pl.ds(base, _WROWS)], idx_v)
        for src_h, dst_h in ((x1_h, ox1), (y1_h, oy1), (x2_h, ox2),
                             (y2_h, oy2), (s_h, os_)):
            pltpu.sync_copy(src_h.at[pl.ds(base, _WROWS)], val_v)
            for r in range(_WROWS):
                pltpu.async_copy(val_v.at[r], dst_h.at[idx_v.at[r]],
                                 sem).wait()

    return permute(rank2d, x1, y1, x2, y2, s)


def _iou_tile(x1r, y1r, x2r, y2r, x1c, y1c, x2c, y2c):
    ix1 = jnp.maximum(x1r, x1c)
    iy1 = jnp.maximum(y1r, y1c)
    ix2 = jnp.minimum(x2r, x2c)
    iy2 = jnp.minimum(y2r, y2c)
    inter = jnp.maximum(ix2 - ix1, 0.0) * jnp.maximum(iy2 - iy1, 0.0)
    ar = (x2r - x1r) * (y2r - y1r)
    ac = (x2c - x1c) * (y2c - y1c)
    union = ar + ac - inter
    return inter / jnp.maximum(union, 1e-9)


def _tri_mask(i, j):
    rr = i * _TILE + lax.broadcasted_iota(jnp.int32, (_TILE, _TILE), 0)
    cc = j * _TILE + lax.broadcasted_iota(jnp.int32, (_TILE, _TILE), 1)
    return rr < cc


def _comp_body(x1r, y1r, x2r, y2r, x1c, y1c, x2c, y2c, comp_ref):
    j = pl.program_id(0)
    i = pl.program_id(1)

    @pl.when(i == 0)
    def _init():
        comp_ref[...] = jnp.zeros_like(comp_ref[...])

    @pl.when(i <= j)
    def _acc():
        iou = _iou_tile(x1r[...], y1r[...], x2r[...], y2r[...],
                        x1c[...], y1c[...], x2c[...], y2c[...])
        iou = jnp.where(_tri_mask(i, j), iou, 0.0)
        comp_ref[...] = jnp.maximum(comp_ref[...],
                                    jnp.max(iou, axis=0, keepdims=True))


def _decay_body(x1r, y1r, x2r, y2r, x1c, y1c, x2c, y2c,
                comp_r, s_c, out_ref):
    j = pl.program_id(0)
    i = pl.program_id(1)

    @pl.when(i == 0)
    def _init():
        out_ref[...] = jnp.zeros_like(out_ref[...])

    @pl.when(i <= j)
    def _acc():
        iou = _iou_tile(x1r[...], y1r[...], x2r[...], y2r[...],
                        x1c[...], y1c[...], x2c[...], y2c[...])
        cr = comp_r[...]
        val = iou * iou - cr * cr
        val = jnp.where(_tri_mask(i, j), val, 0.0)
        out_ref[...] = jnp.maximum(out_ref[...],
                                   jnp.max(val, axis=0, keepdims=True))

    @pl.when(i == _NB - 1)
    def _fin():
        ns = s_c[...] * jnp.exp(-out_ref[...] / _SIGMA)
        out_ref[...] = jnp.where(ns >= _THRESH, ns, 0.0)


_row = pl.BlockSpec((_TILE, 1), lambda j, i: (i, 0))
_col = pl.BlockSpec((1, _TILE), lambda j, i: (0, j))
_out = pl.BlockSpec((1, _TILE), lambda j, i: (0, j))


def _sweep_comp(x1s, y1s, x2s, y2s):
    r = lambda a: a.reshape(_NP, 1)
    c = lambda a: a.reshape(1, _NP)
    return pl.pallas_call(
        _comp_body,
        grid=(_NB, _NB),
        in_specs=[_row] * 4 + [_col] * 4,
        out_specs=_out,
        out_shape=jax.ShapeDtypeStruct((1, _NP), jnp.float32),
    )(r(x1s), r(y1s), r(x2s), r(y2s), c(x1s), c(y1s), c(x2s), c(y2s))


def _sweep_decay(x1s, y1s, x2s, y2s, comp, ss):
    r = lambda a: a.reshape(_NP, 1)
    c = lambda a: a.reshape(1, _NP)
    return pl.pallas_call(
        _decay_body,
        grid=(_NB, _NB),
        in_specs=[_row] * 4 + [_col] * 4 + [_row, _col],
        out_specs=_out,
        out_shape=jax.ShapeDtypeStruct((1, _NP), jnp.float32),
    )(r(x1s), r(y1s), r(x2s), r(y2s), c(x1s), c(y1s), c(x2s), c(y2s),
      comp.reshape(_NP, 1), c(ss))


def kernel(boxes, scores):
    s_pad = jnp.concatenate(
        [scores, jnp.full((_NP - _N,), -1.0, jnp.float32)])
    b_pad = jnp.concatenate(
        [boxes, jnp.zeros((_NP - _N, 4), jnp.float32)], axis=0)

    rank = _compute_rank(s_pad)

    v = lambda a: a.reshape(_ROWS, _COLS)
    x1s, y1s, x2s, y2s, ss = _sc_permute(
        v(rank.reshape(_NP)), v(b_pad[:, 0]), v(b_pad[:, 1]),
        v(b_pad[:, 2]), v(b_pad[:, 3]), v(s_pad))

    comp = _sweep_comp(x1s, y1s, x2s, y2s)
    out = _sweep_decay(x1s, y1s, x2s, y2s, comp, ss)
    return out.reshape(_NP)[:_N]


# speedup vs baseline: 1.0774x; 1.0774x over previous
import functools

import jax
import jax.numpy as jnp
from jax import lax
from jax.experimental import pallas as pl
from jax.experimental.pallas import tpu as pltpu
from jax.experimental.pallas import tpu_sc as plsc

_SIGMA = 0.5
_THRESH = 0.05
_N = 5000
_NP = 5120
_TILE = 512
_NB = _NP // _TILE
_RT = 256
_ROWS = 64
_COLS = 80
_WROWS = 2


def _rank_body(s_col_ref, s_row_ref, rank_ref):
    i = pl.program_id(0)
    s_i = s_col_ref[...]
    s_j = s_row_ref[...]
    gi = i * _RT + lax.broadcasted_iota(jnp.int32, (_RT, _NP), 0)
    gj = lax.broadcasted_iota(jnp.int32, (_RT, _NP), 1)
    before = (s_j > s_i) | ((s_j == s_i) & (gj < gi))
    rank_ref[...] = jnp.sum(before.astype(jnp.int32), axis=1, keepdims=True)


def _compute_rank(s_pad):
    return pl.pallas_call(
        _rank_body,
        grid=(_NP // _RT,),
        in_specs=[
            pl.BlockSpec((_RT, 1), lambda i: (i, 0)),
            pl.BlockSpec((1, _NP), lambda i: (0, 0)),
        ],
        out_specs=pl.BlockSpec((_RT, 1), lambda i: (i, 0)),
        out_shape=jax.ShapeDtypeStruct((_NP, 1), jnp.int32),
    )(s_pad.reshape(_NP, 1), s_pad.reshape(1, _NP))


def _sc_permute(rank2d, vals3d):
    mesh = plsc.VectorSubcoreMesh(core_axis_name="c", subcore_axis_name="s")

    @functools.partial(
        pl.kernel,
        mesh=mesh,
        out_type=[jax.ShapeDtypeStruct((_NP,), jnp.float32)] * 5,
        scratch_types=[
            pltpu.VMEM((_WROWS, _COLS), jnp.int32),
            pltpu.VMEM((_WROWS, 5, _COLS), jnp.float32),
            pltpu.SemaphoreType.DMA,
        ],
    )
    def permute(rank_h, vals_h, ox1, oy1, ox2, oy2, os_, idx_v, val_v, sem):
        w = lax.axis_index("s") * 2 + lax.axis_index("c")
        base = w * _WROWS
        pltpu.sync_copy(rank_h.at[pl.ds(base, _WROWS)], idx_v)
        pltpu.sync_copy(vals_h.at[pl.ds(base, _WROWS)], val_v)
        cps = []
        for k, dst_h in enumerate((ox1, oy1, ox2, oy2, os_)):
            for r in range(_WROWS):
                cps.append(pltpu.async_copy(
                    val_v.at[r, k], dst_h.at[idx_v.at[r]], sem))
        for cp in cps:
            cp.wait()

    return permute(rank2d, vals3d)


def _iou_tile(x1r, y1r, x2r, y2r, x1c, y1c, x2c, y2c):
    ix1 = jnp.maximum(x1r, x1c)
    iy1 = jnp.maximum(y1r, y1c)
    ix2 = jnp.minimum(x2r, x2c)
    iy2 = jnp.minimum(y2r, y2c)
    inter = jnp.maximum(ix2 - ix1, 0.0) * jnp.maximum(iy2 - iy1, 0.0)
    ar = (x2r - x1r) * (y2r - y1r)
    ac = (x2c - x1c) * (y2c - y1c)
    union = ar + ac - inter
    return inter / jnp.maximum(union, 1e-9)


def _tri_mask(i, j):
    rr = i * _TILE + lax.broadcasted_iota(jnp.int32, (_TILE, _TILE), 0)
    cc = j * _TILE + lax.broadcasted_iota(jnp.int32, (_TILE, _TILE), 1)
    return rr < cc


def _comp_body(x1r, y1r, x2r, y2r, x1c, y1c, x2c, y2c, comp_ref):
    j = pl.program_id(0)
    i = pl.program_id(1)

    @pl.when(i == 0)
    def _init():
        comp_ref[...] = jnp.zeros_like(comp_ref[...])

    @pl.when(i <= j)
    def _acc():
        iou = _iou_tile(x1r[...], y1r[...], x2r[...], y2r[...],
                        x1c[...], y1c[...], x2c[...], y2c[...])
        iou = jnp.where(_tri_mask(i, j), iou, 0.0)
        comp_ref[...] = jnp.maximum(comp_ref[...],
                                    jnp.max(iou, axis=0, keepdims=True))


def _decay_body(x1r, y1r, x2r, y2r, x1c, y1c, x2c, y2c,
                comp_r, s_c, out_ref):
    j = pl.program_id(0)
    i = pl.program_id(1)

    @pl.when(i == 0)
    def _init():
        out_ref[...] = jnp.zeros_like(out_ref[...])

    @pl.when(i <= j)
    def _acc():
        iou = _iou_tile(x1r[...], y1r[...], x2r[...], y2r[...],
                        x1c[...], y1c[...], x2c[...], y2c[...])
        cr = comp_r[...]
        val = iou * iou - cr * cr
        val = jnp.where(_tri_mask(i, j), val, 0.0)
        out_ref[...] = jnp.maximum(out_ref[...],
                                   jnp.max(val, axis=0, keepdims=True))

    @pl.when(i == _NB - 1)
    def _fin():
        ns = s_c[...] * jnp.exp(-out_ref[...] / _SIGMA)
        out_ref[...] = jnp.where(ns >= _THRESH, ns, 0.0)


_row = pl.BlockSpec((_TILE, 1), lambda j, i: (i, 0))
_col = pl.BlockSpec((1, _TILE), lambda j, i: (0, j))
_out = pl.BlockSpec((1, _TILE), lambda j, i: (0, j))


def _sweep_comp(x1s, y1s, x2s, y2s):
    r = lambda a: a.reshape(_NP, 1)
    c = lambda a: a.reshape(1, _NP)
    return pl.pallas_call(
        _comp_body,
        grid=(_NB, _NB),
        in_specs=[_row] * 4 + [_col] * 4,
        out_specs=_out,
        out_shape=jax.ShapeDtypeStruct((1, _NP), jnp.float32),
    )(r(x1s), r(y1s), r(x2s), r(y2s), c(x1s), c(y1s), c(x2s), c(y2s))


def _sweep_decay(x1s, y1s, x2s, y2s, comp, ss):
    r = lambda a: a.reshape(_NP, 1)
    c = lambda a: a.reshape(1, _NP)
    return pl.pallas_call(
        _decay_body,
        grid=(_NB, _NB),
        in_specs=[_row] * 4 + [_col] * 4 + [_row, _col],
        out_specs=_out,
        out_shape=jax.ShapeDtypeStruct((1, _NP), jnp.float32),
    )(r(x1s), r(y1s), r(x2s), r(y2s), c(x1s), c(y1s), c(x2s), c(y2s),
      comp.reshape(_NP, 1), c(ss))


def kernel(boxes, scores):
    s_pad = jnp.concatenate(
        [scores, jnp.full((_NP - _N,), -1.0, jnp.float32)])
    b_pad = jnp.concatenate(
        [boxes, jnp.zeros((_NP - _N, 4), jnp.float32)], axis=0)

    rank = _compute_rank(s_pad)

    vals = jnp.stack(
        [b_pad[:, 0], b_pad[:, 1], b_pad[:, 2], b_pad[:, 3], s_pad],
        axis=0)
    vals = vals.reshape(5, _ROWS, _COLS).transpose(1, 0, 2)
    x1s, y1s, x2s, y2s, ss = _sc_permute(
        rank.reshape(_ROWS, _COLS), vals)

    comp = _sweep_comp(x1s, y1s, x2s, y2s)
    out = _sweep_decay(x1s, y1s, x2s, y2s, comp, ss)
    return out.reshape(_NP)[:_N]


# speedup vs baseline: 1.7432x; 1.6180x over previous
import functools

import numpy as np

import jax
import jax.numpy as jnp
from jax import lax
from jax.experimental import pallas as pl
from jax.experimental.pallas import tpu as pltpu
from jax.experimental.pallas import tpu_sc as plsc

_SIGMA = 0.5
_THRESH = 0.05
_N = 5000
_NP = 5120
_TILE = 512
_NB = _NP // _TILE
_RT = 256
_ROWS = 64
_COLS = 80
_WROWS = 2


def _rank_body(s_col_ref, s_row_ref, rank_ref):
    i = pl.program_id(0)
    s_i = s_col_ref[...]
    s_j = s_row_ref[...]
    gi = i * _RT + lax.broadcasted_iota(jnp.int32, (_RT, _NP), 0)
    gj = lax.broadcasted_iota(jnp.int32, (_RT, _NP), 1)
    before = (s_j > s_i) | ((s_j == s_i) & (gj < gi))
    rank_ref[...] = jnp.sum(before.astype(jnp.int32), axis=1, keepdims=True)


def _compute_rank(s_pad):
    return pl.pallas_call(
        _rank_body,
        grid=(_NP // _RT,),
        in_specs=[
            pl.BlockSpec((_RT, 1), lambda i: (i, 0)),
            pl.BlockSpec((1, _NP), lambda i: (0, 0)),
        ],
        out_specs=pl.BlockSpec((_RT, 1), lambda i: (i, 0)),
        out_shape=jax.ShapeDtypeStruct((_NP, 1), jnp.int32),
    )(s_pad.reshape(_NP, 1), s_pad.reshape(1, _NP))


def _sc_permute(rank2d, vals3d):
    mesh = plsc.VectorSubcoreMesh(core_axis_name="c", subcore_axis_name="s")

    @functools.partial(
        pl.kernel,
        mesh=mesh,
        out_type=[jax.ShapeDtypeStruct((_NP,), jnp.float32)] * 5,
        scratch_types=[
            pltpu.VMEM((_WROWS, _COLS), jnp.int32),
            pltpu.VMEM((_WROWS, 5, _COLS), jnp.float32),
            pltpu.SemaphoreType.DMA,
        ],
    )
    def permute(rank_h, vals_h, ox1, oy1, ox2, oy2, os_, idx_v, val_v, sem):
        w = lax.axis_index("s") * 2 + lax.axis_index("c")
        base = w * _WROWS
        pltpu.sync_copy(rank_h.at[pl.ds(base, _WROWS)], idx_v)
        pltpu.sync_copy(vals_h.at[pl.ds(base, _WROWS)], val_v)
        cps = []
        for k, dst_h in enumerate((ox1, oy1, ox2, oy2, os_)):
            for r in range(_WROWS):
                cps.append(pltpu.async_copy(
                    val_v.at[r, k], dst_h.at[idx_v.at[r]], sem))
        for cp in cps:
            cp.wait()

    return permute(rank2d, vals3d)


def _iou_tile(x1r, y1r, x2r, y2r, x1c, y1c, x2c, y2c):
    ix1 = jnp.maximum(x1r, x1c)
    iy1 = jnp.maximum(y1r, y1c)
    ix2 = jnp.minimum(x2r, x2c)
    iy2 = jnp.minimum(y2r, y2c)
    inter = jnp.maximum(ix2 - ix1, 0.0) * jnp.maximum(iy2 - iy1, 0.0)
    ar = (x2r - x1r) * (y2r - y1r)
    ac = (x2c - x1c) * (y2c - y1c)
    union = ar + ac - inter
    return inter / jnp.maximum(union, 1e-9)


def _tri_mask(i, j):
    rr = i * _TILE + lax.broadcasted_iota(jnp.int32, (_TILE, _TILE), 0)
    cc = j * _TILE + lax.broadcasted_iota(jnp.int32, (_TILE, _TILE), 1)
    return rr < cc


_TRI = [(i, j) for j in range(_NB) for i in range(j + 1)]
_I_ARR = np.array([t[0] for t in _TRI], np.int32)
_J_ARR = np.array([t[1] for t in _TRI], np.int32)


def _fused_body(i_ref, j_ref, x1r, y1r, x2r, y2r, x1c, y1c, x2c, y2c,
                s_c, out_ref, comp_row, comp_col):
    t = pl.program_id(0)
    i = i_ref[t]
    j = j_ref[t]

    @pl.when(t == 0)
    def _init_comp():
        comp_row[...] = jnp.zeros_like(comp_row[...])

    @pl.when(i == 0)
    def _init_m():
        out_ref[...] = jnp.zeros_like(out_ref[...])

    iou = _iou_tile(x1r[...], y1r[...], x2r[...], y2r[...],
                    x1c[...], y1c[...], x2c[...], y2c[...])

    @pl.when(i < j)
    def _off_diag():
        comp_row[j] = jnp.maximum(comp_row[j],
                                  jnp.max(iou, axis=0, keepdims=True))
        cr = comp_col[i]
        val = iou * iou - cr * cr
        out_ref[...] = jnp.maximum(out_ref[...],
                                   jnp.max(val, axis=0, keepdims=True))

    @pl.when(i == j)
    def _diag():
        rr = lax.broadcasted_iota(jnp.int32, (_TILE, _TILE), 0)
        cc = lax.broadcasted_iota(jnp.int32, (_TILE, _TILE), 1)
        mask = rr < cc
        iou_m = jnp.where(mask, iou, 0.0)
        comp_new = jnp.maximum(comp_row[j],
                               jnp.max(iou_m, axis=0, keepdims=True))
        comp_col[j] = jnp.transpose(comp_new, (1, 0))
        cr = comp_col[j]
        val = jnp.where(mask, iou * iou - cr * cr, 0.0)
        m = jnp.maximum(out_ref[...], jnp.max(val, axis=0, keepdims=True))
        ns = s_c[...] * jnp.exp(m * (-1.0 / _SIGMA))
        out_ref[...] = jnp.where(ns >= _THRESH, ns, 0.0)


def _fused_sweep(x1s, y1s, x2s, y2s, ss):
    r = lambda a: a.reshape(_NP, 1)
    c = lambda a: a.reshape(1, _NP)
    row = pl.BlockSpec((_TILE, 1), lambda t, i_ref, j_ref: (i_ref[t], 0))
    col = pl.BlockSpec((1, _TILE), lambda t, i_ref, j_ref: (0, j_ref[t]))
    grid_spec = pltpu.PrefetchScalarGridSpec(
        num_scalar_prefetch=2,
        grid=(len(_TRI),),
        in_specs=[row] * 4 + [col] * 4 + [col],
        out_specs=col,
        scratch_shapes=[
            pltpu.VMEM((_NB, 1, _TILE), jnp.float32),
            pltpu.VMEM((_NB, _TILE, 1), jnp.float32),
        ],
    )
    return pl.pallas_call(
        _fused_body,
        grid_spec=grid_spec,
        out_shape=jax.ShapeDtypeStruct((1, _NP), jnp.float32),
    )(jnp.asarray(_I_ARR), jnp.asarray(_J_ARR),
      r(x1s), r(y1s), r(x2s), r(y2s), c(x1s), c(y1s), c(x2s), c(y2s), c(ss))


def kernel(boxes, scores):
    s_pad = jnp.concatenate(
        [scores, jnp.full((_NP - _N,), -1.0, jnp.float32)])
    b_pad = jnp.concatenate(
        [boxes, jnp.zeros((_NP - _N, 4), jnp.float32)], axis=0)

    rank = _compute_rank(s_pad)

    vals = jnp.stack(
        [b_pad[:, 0], b_pad[:, 1], b_pad[:, 2], b_pad[:, 3], s_pad],
        axis=0)
    vals = vals.reshape(5, _ROWS, _COLS).transpose(1, 0, 2)
    x1s, y1s, x2s, y2s, ss = _sc_permute(
        rank.reshape(_ROWS, _COLS), vals)

    out = _fused_sweep(x1s, y1s, x2s, y2s, ss)
    return out.reshape(_NP)[:_N]


# speedup vs baseline: 1.8755x; 1.0759x over previous
import functools

import numpy as np

import jax
import jax.numpy as jnp
from jax import lax
from jax.experimental import pallas as pl
from jax.experimental.pallas import tpu as pltpu
from jax.experimental.pallas import tpu_sc as plsc

_SIGMA = 0.5
_THRESH = 0.05
_N = 5000
_NP = 5120
_TILE = 1024
_NB = _NP // _TILE
_RT = 512
_ROWS = 64
_COLS = 80
_WROWS = 2


def _rank_body(s_col_ref, s_row_ref, rank_ref):
    i = pl.program_id(0)
    s_i = s_col_ref[...]
    s_j = s_row_ref[...]
    gi = i * _RT + lax.broadcasted_iota(jnp.int32, (_RT, _NP), 0)
    gj = lax.broadcasted_iota(jnp.int32, (_RT, _NP), 1)
    before = (s_j > s_i) | ((s_j == s_i) & (gj < gi))
    rank_ref[...] = jnp.sum(before.astype(jnp.int32), axis=1, keepdims=True)


def _compute_rank(s_pad):
    return pl.pallas_call(
        _rank_body,
        grid=(_NP // _RT,),
        in_specs=[
            pl.BlockSpec((_RT, 1), lambda i: (i, 0)),
            pl.BlockSpec((1, _NP), lambda i: (0, 0)),
        ],
        out_specs=pl.BlockSpec((_RT, 1), lambda i: (i, 0)),
        out_shape=jax.ShapeDtypeStruct((_NP, 1), jnp.int32),
    )(s_pad.reshape(_NP, 1), s_pad.reshape(1, _NP))


def _sc_permute(rank2d, vals3d):
    mesh = plsc.VectorSubcoreMesh(core_axis_name="c", subcore_axis_name="s")

    @functools.partial(
        pl.kernel,
        mesh=mesh,
        out_type=[jax.ShapeDtypeStruct((_NP,), jnp.float32)] * 5,
        scratch_types=[
            pltpu.VMEM((_WROWS, _COLS), jnp.int32),
            pltpu.VMEM((_WROWS, 5, _COLS), jnp.float32),
            pltpu.SemaphoreType.DMA,
        ],
    )
    def permute(rank_h, vals_h, ox1, oy1, ox2, oy2, os_, idx_v, val_v, sem):
        w = lax.axis_index("s") * 2 + lax.axis_index("c")
        base = w * _WROWS
        stg = [pltpu.async_copy(rank_h.at[pl.ds(base, _WROWS)], idx_v, sem),
               pltpu.async_copy(vals_h.at[pl.ds(base, _WROWS)], val_v, sem)]
        for cp in stg:
            cp.wait()
        cps = []
        for k, dst_h in enumerate((ox1, oy1, ox2, oy2, os_)):
            for r in range(_WROWS):
                cps.append(pltpu.async_copy(
                    val_v.at[r, k], dst_h.at[idx_v.at[r]], sem))
        for cp in cps:
            cp.wait()

    return permute(rank2d, vals3d)


def _iou_tile(x1r, y1r, x2r, y2r, x1c, y1c, x2c, y2c):
    ix1 = jnp.maximum(x1r, x1c)
    iy1 = jnp.maximum(y1r, y1c)
    ix2 = jnp.minimum(x2r, x2c)
    iy2 = jnp.minimum(y2r, y2c)
    inter = jnp.maximum(ix2 - ix1, 0.0) * jnp.maximum(iy2 - iy1, 0.0)
    ar = (x2r - x1r) * (y2r - y1r)
    ac = (x2c - x1c) * (y2c - y1c)
    union = ar + ac - inter
    return inter * pl.reciprocal(jnp.maximum(union, 1e-9), approx=True)


def _tri_mask(i, j):
    rr = i * _TILE + lax.broadcasted_iota(jnp.int32, (_TILE, _TILE), 0)
    cc = j * _TILE + lax.broadcasted_iota(jnp.int32, (_TILE, _TILE), 1)
    return rr < cc


_TRI = [(i, j) for j in range(_NB) for i in range(j + 1)]
_I_ARR = np.array([t[0] for t in _TRI], np.int32)
_J_ARR = np.array([t[1] for t in _TRI], np.int32)


def _fused_body(i_ref, j_ref, x1r, y1r, x2r, y2r, x1c, y1c, x2c, y2c,
                s_c, out_ref, comp_row, comp_col):
    t = pl.program_id(0)
    i = i_ref[t]
    j = j_ref[t]

    @pl.when(t == 0)
    def _init_comp():
        comp_row[...] = jnp.zeros_like(comp_row[...])

    @pl.when(i == 0)
    def _init_m():
        out_ref[...] = jnp.zeros_like(out_ref[...])

    iou = _iou_tile(x1r[...], y1r[...], x2r[...], y2r[...],
                    x1c[...], y1c[...], x2c[...], y2c[...])

    @pl.when(i < j)
    def _off_diag():
        comp_row[j] = jnp.maximum(comp_row[j],
                                  jnp.max(iou, axis=0, keepdims=True))
        cr = comp_col[i]
        val = iou * iou - cr * cr
        out_ref[...] = jnp.maximum(out_ref[...],
                                   jnp.max(val, axis=0, keepdims=True))

    @pl.when(i == j)
    def _diag():
        rr = lax.broadcasted_iota(jnp.int32, (_TILE, _TILE), 0)
        cc = lax.broadcasted_iota(jnp.int32, (_TILE, _TILE), 1)
        mask = rr < cc
        iou_m = jnp.where(mask, iou, 0.0)
        comp_new = jnp.maximum(comp_row[j],
                               jnp.max(iou_m, axis=0, keepdims=True))
        comp_col[j] = jnp.transpose(comp_new, (1, 0))
        cr = comp_col[j]
        val = jnp.where(mask, iou * iou - cr * cr, 0.0)
        m = jnp.maximum(out_ref[...], jnp.max(val, axis=0, keepdims=True))
        ns = s_c[...] * jnp.exp(m * (-1.0 / _SIGMA))
        out_ref[...] = jnp.where(ns >= _THRESH, ns, 0.0)


def _fused_sweep(x1s, y1s, x2s, y2s, ss):
    r = lambda a: a.reshape(_NP, 1)
    c = lambda a: a.reshape(1, _NP)
    row = pl.BlockSpec((_TILE, 1), lambda t, i_ref, j_ref: (i_ref[t], 0))
    col = pl.BlockSpec((1, _TILE), lambda t, i_ref, j_ref: (0, j_ref[t]))
    grid_spec = pltpu.PrefetchScalarGridSpec(
        num_scalar_prefetch=2,
        grid=(len(_TRI),),
        in_specs=[row] * 4 + [col] * 4 + [col],
        out_specs=col,
        scratch_shapes=[
            pltpu.VMEM((_NB, 1, _TILE), jnp.float32),
            pltpu.VMEM((_NB, _TILE, 1), jnp.float32),
        ],
    )
    return pl.pallas_call(
        _fused_body,
        grid_spec=grid_spec,
        out_shape=jax.ShapeDtypeStruct((1, _NP), jnp.float32),
    )(jnp.asarray(_I_ARR), jnp.asarray(_J_ARR),
      r(x1s), r(y1s), r(x2s), r(y2s), c(x1s), c(y1s), c(x2s), c(y2s), c(ss))


def kernel(boxes, scores):
    s_pad = jnp.concatenate(
        [scores, jnp.full((_NP - _N,), -1.0, jnp.float32)])
    b_pad = jnp.concatenate(
        [boxes, jnp.zeros((_NP - _N, 4), jnp.float32)], axis=0)

    rank = _compute_rank(s_pad)

    vals = jnp.stack(
        [b_pad[:, 0], b_pad[:, 1], b_pad[:, 2], b_pad[:, 3], s_pad],
        axis=0)
    vals = vals.reshape(5, _ROWS, _COLS).transpose(1, 0, 2)
    x1s, y1s, x2s, y2s, ss = _sc_permute(
        rank.reshape(_ROWS, _COLS), vals)

    out = _fused_sweep(x1s, y1s, x2s, y2s, ss)
    return out.reshape(_NP)[:_N]


# speedup vs baseline: 2.9835x; 1.5908x over previous
import functools

import numpy as np

import jax
import jax.numpy as jnp
from jax import lax
from jax.experimental import pallas as pl
from jax.experimental.pallas import tpu as pltpu
from jax.experimental.pallas import tpu_sc as plsc

_SIGMA = 0.5
_THRESH = 0.05
_N = 5000
_NP = 5120
_TILE = 1024
_NB = _NP // _TILE
_RT = 512
_ROWS = 64
_COLS = 80
_WROWS = 2


def _rank_body(s_col_ref, s_row_ref, rank_ref):
    i = pl.program_id(0)
    s_i = s_col_ref[...]
    s_j = s_row_ref[...]
    gi = i * _RT + lax.broadcasted_iota(jnp.int32, (_RT, _NP), 0)
    gj = lax.broadcasted_iota(jnp.int32, (_RT, _NP), 1)
    before = (s_j > s_i) | ((s_j == s_i) & (gj < gi))
    rank_ref[...] = jnp.sum(before.astype(jnp.int32), axis=1, keepdims=True)


def _compute_rank(s_pad):
    return pl.pallas_call(
        _rank_body,
        grid=(_NP // _RT,),
        in_specs=[
            pl.BlockSpec((_RT, 1), lambda i: (i, 0)),
            pl.BlockSpec((1, _NP), lambda i: (0, 0)),
        ],
        out_specs=pl.BlockSpec((_RT, 1), lambda i: (i, 0)),
        out_shape=jax.ShapeDtypeStruct((_NP, 1), jnp.int32),
    )(s_pad.reshape(_NP, 1), s_pad.reshape(1, _NP))


def _sc_permute(rank2d, vals3d):
    mesh = plsc.VectorSubcoreMesh(core_axis_name="c", subcore_axis_name="s")

    @functools.partial(
        pl.kernel,
        mesh=mesh,
        out_type=jax.ShapeDtypeStruct((_NP, 8), jnp.float32),
        scratch_types=[
            pltpu.VMEM((_WROWS, _COLS), jnp.int32),
            pltpu.VMEM((_WROWS, _COLS, 8), jnp.float32),
            pltpu.SemaphoreType.DMA,
        ],
        compiler_params=pltpu.CompilerParams(use_tc_tiling_on_sc=False),
    )
    def permute(rank_h, vals_h, out_h, idx_v, val_v, sem):
        w = lax.axis_index("s") * 2 + lax.axis_index("c")
        base = w * _WROWS
        stg = [pltpu.async_copy(rank_h.at[pl.ds(base, _WROWS)], idx_v, sem),
               pltpu.async_copy(vals_h.at[pl.ds(base, _WROWS)], val_v, sem)]
        for cp in stg:
            cp.wait()
        cps = [pltpu.async_copy(val_v.at[r], out_h.at[idx_v.at[r]], sem)
               for r in range(_WROWS)]
        for cp in cps:
            cp.wait()

    return permute(rank2d, vals3d)


def _iou_tile(x1r, y1r, x2r, y2r, x1c, y1c, x2c, y2c):
    ix1 = jnp.maximum(x1r, x1c)
    iy1 = jnp.maximum(y1r, y1c)
    ix2 = jnp.minimum(x2r, x2c)
    iy2 = jnp.minimum(y2r, y2c)
    inter = jnp.maximum(ix2 - ix1, 0.0) * jnp.maximum(iy2 - iy1, 0.0)
    ar = (x2r - x1r) * (y2r - y1r)
    ac = (x2c - x1c) * (y2c - y1c)
    union = ar + ac - inter
    return inter * pl.reciprocal(union, approx=True)


def _tri_mask(i, j):
    rr = i * _TILE + lax.broadcasted_iota(jnp.int32, (_TILE, _TILE), 0)
    cc = j * _TILE + lax.broadcasted_iota(jnp.int32, (_TILE, _TILE), 1)
    return rr < cc


_TRI = [(i, j) for j in range(_NB) for i in range(j + 1)]
_I_ARR = np.array([t[0] for t in _TRI], np.int32)
_J_ARR = np.array([t[1] for t in _TRI], np.int32)


def _fused_body(i_ref, j_ref, x1r, y1r, x2r, y2r, x1c, y1c, x2c, y2c,
                s_c, out_ref, comp_row, comp_col):
    t = pl.program_id(0)
    i = i_ref[t]
    j = j_ref[t]

    @pl.when(t == 0)
    def _init_comp():
        comp_row[...] = jnp.zeros_like(comp_row[...])

    @pl.when(i == 0)
    def _init_m():
        out_ref[...] = jnp.zeros_like(out_ref[...])

    iou = _iou_tile(x1r[...], y1r[...], x2r[...], y2r[...],
                    x1c[...], y1c[...], x2c[...], y2c[...])

    @pl.when(i < j)
    def _off_diag():
        comp_row[j] = jnp.maximum(comp_row[j],
                                  jnp.max(iou, axis=0, keepdims=True))
        cr = comp_col[i]
        val = iou * iou - cr * cr
        out_ref[...] = jnp.maximum(out_ref[...],
                                   jnp.max(val, axis=0, keepdims=True))

    @pl.when(i == j)
    def _diag():
        rr = lax.broadcasted_iota(jnp.int32, (_TILE, _TILE), 0)
        cc = lax.broadcasted_iota(jnp.int32, (_TILE, _TILE), 1)
        mask = rr < cc
        iou_m = jnp.where(mask, iou, 0.0)
        comp_new = jnp.maximum(comp_row[j],
                               jnp.max(iou_m, axis=0, keepdims=True))
        comp_col[j] = jnp.transpose(comp_new, (1, 0))
        cr = comp_col[j]
        val = jnp.where(mask, iou * iou - cr * cr, 0.0)
        m = jnp.maximum(out_ref[...], jnp.max(val, axis=0, keepdims=True))
        ns = s_c[...] * jnp.exp(m * (-1.0 / _SIGMA))
        out_ref[...] = jnp.where(ns >= _THRESH, ns, 0.0)


def _fused_sweep(x1s, y1s, x2s, y2s, ss):
    r = lambda a: a.reshape(_NP, 1)
    c = lambda a: a.reshape(1, _NP)
    row = pl.BlockSpec((_TILE, 1), lambda t, i_ref, j_ref: (i_ref[t], 0))
    col = pl.BlockSpec((1, _TILE), lambda t, i_ref, j_ref: (0, j_ref[t]))
    grid_spec = pltpu.PrefetchScalarGridSpec(
        num_scalar_prefetch=2,
        grid=(len(_TRI),),
        in_specs=[row] * 4 + [col] * 4 + [col],
        out_specs=col,
        scratch_shapes=[
            pltpu.VMEM((_NB, 1, _TILE), jnp.float32),
            pltpu.VMEM((_NB, _TILE, 1), jnp.float32),
        ],
    )
    return pl.pallas_call(
        _fused_body,
        grid_spec=grid_spec,
        out_shape=jax.ShapeDtypeStruct((1, _NP), jnp.float32),
    )(jnp.asarray(_I_ARR), jnp.asarray(_J_ARR),
      r(x1s), r(y1s), r(x2s), r(y2s), c(x1s), c(y1s), c(x2s), c(y2s), c(ss))


def kernel(boxes, scores):
    s_pad = jnp.concatenate(
        [scores, jnp.full((_NP - _N,), -1.0, jnp.float32)])
    b_pad = jnp.concatenate(
        [boxes, jnp.zeros((_NP - _N, 4), jnp.float32)], axis=0)

    rank = _compute_rank(s_pad)

    vals = jnp.concatenate(
        [b_pad, s_pad.reshape(_NP, 1),
         jnp.zeros((_NP, 3), jnp.float32)], axis=1)
    sorted_vals = _sc_permute(
        rank.reshape(_ROWS, _COLS), vals.reshape(_ROWS, _COLS, 8))
    x1s, y1s, x2s, y2s = (sorted_vals[:, 0], sorted_vals[:, 1],
                          sorted_vals[:, 2], sorted_vals[:, 3])
    ss = sorted_vals[:, 4]

    out = _fused_sweep(x1s, y1s, x2s, y2s, ss)
    return out.reshape(_NP)[:_N]


# speedup vs baseline: 3.0678x; 1.0282x over previous
import functools

import numpy as np

import jax
import jax.numpy as jnp
from jax import lax
from jax.experimental import pallas as pl
from jax.experimental.pallas import tpu as pltpu
from jax.experimental.pallas import tpu_sc as plsc

_SIGMA = 0.5
_THRESH = 0.05
_N = 5000
_NP = 5120
_TILE = 1024
_NB = _NP // _TILE
_RT = 512
_ROWS = 64
_COLS = 80
_WROWS = 2


def _rank_body(s_col_ref, s_row_ref, rank_ref):
    i = pl.program_id(0)
    k_i = 2 * jax.lax.bitcast_convert_type(s_col_ref[...], jnp.int32)
    k_j = 2 * jax.lax.bitcast_convert_type(s_row_ref[...], jnp.int32)
    gi = i * _RT + lax.broadcasted_iota(jnp.int32, (_RT, _NP), 0)
    gj = lax.broadcasted_iota(jnp.int32, (_RT, _NP), 1)
    before = (k_j + (gj < gi).astype(jnp.int32)) > k_i
    rank_ref[...] = jnp.sum(before.astype(jnp.int32), axis=1, keepdims=True)


def _compute_rank(s_pad):
    return pl.pallas_call(
        _rank_body,
        grid=(_NP // _RT,),
        in_specs=[
            pl.BlockSpec((_RT, 1), lambda i: (i, 0)),
            pl.BlockSpec((1, _NP), lambda i: (0, 0)),
        ],
        out_specs=pl.BlockSpec((_RT, 1), lambda i: (i, 0)),
        out_shape=jax.ShapeDtypeStruct((_NP, 1), jnp.int32),
    )(s_pad.reshape(_NP, 1), s_pad.reshape(1, _NP))


def _sc_permute(rank2d, vals3d):
    mesh = plsc.VectorSubcoreMesh(core_axis_name="c", subcore_axis_name="s")

    @functools.partial(
        pl.kernel,
        mesh=mesh,
        out_type=jax.ShapeDtypeStruct((_NP, 8), jnp.float32),
        scratch_types=[
            pltpu.VMEM((_WROWS, _COLS), jnp.int32),
            pltpu.VMEM((_WROWS, _COLS, 8), jnp.float32),
            pltpu.SemaphoreType.DMA,
        ],
        compiler_params=pltpu.CompilerParams(use_tc_tiling_on_sc=False),
    )
    def permute(rank_h, vals_h, out_h, idx_v, val_v, sem):
        w = lax.axis_index("s") * 2 + lax.axis_index("c")
        base = w * _WROWS
        stg = [pltpu.async_copy(rank_h.at[pl.ds(base, _WROWS)], idx_v, sem),
               pltpu.async_copy(vals_h.at[pl.ds(base, _WROWS)], val_v, sem)]
        for cp in stg:
            cp.wait()
        cps = [pltpu.async_copy(val_v.at[r], out_h.at[idx_v.at[r]], sem)
               for r in range(_WROWS)]
        for cp in cps:
            cp.wait()

    return permute(rank2d, vals3d)


def _iou_tile(x1r, y1r, x2r, y2r, x1c, y1c, x2c, y2c):
    ix1 = jnp.maximum(x1r, x1c)
    iy1 = jnp.maximum(y1r, y1c)
    ix2 = jnp.minimum(x2r, x2c)
    iy2 = jnp.minimum(y2r, y2c)
    inter = jnp.maximum(ix2 - ix1, 0.0) * jnp.maximum(iy2 - iy1, 0.0)
    ar = (x2r - x1r) * (y2r - y1r)
    ac = (x2c - x1c) * (y2c - y1c)
    union = ar + ac - inter
    return inter * pl.reciprocal(union, approx=True)


def _tri_mask(i, j):
    rr = i * _TILE + lax.broadcasted_iota(jnp.int32, (_TILE, _TILE), 0)
    cc = j * _TILE + lax.broadcasted_iota(jnp.int32, (_TILE, _TILE), 1)
    return rr < cc


_TRI = [(i, j) for j in range(_NB) for i in range(j + 1)]
_I_ARR = np.array([t[0] for t in _TRI], np.int32)
_J_ARR = np.array([t[1] for t in _TRI], np.int32)


def _fused_body(i_ref, j_ref, x1r, y1r, x2r, y2r, x1c, y1c, x2c, y2c,
                s_c, out_ref, comp_row, comp_col):
    t = pl.program_id(0)
    i = i_ref[t]
    j = j_ref[t]

    @pl.when(t == 0)
    def _init_comp():
        comp_row[...] = jnp.zeros_like(comp_row[...])

    @pl.when(i == 0)
    def _init_m():
        out_ref[...] = jnp.zeros_like(out_ref[...])

    iou = _iou_tile(x1r[...], y1r[...], x2r[...], y2r[...],
                    x1c[...], y1c[...], x2c[...], y2c[...])

    @pl.when(i < j)
    def _off_diag():
        comp_row[j] = jnp.maximum(comp_row[j],
                                  jnp.max(iou, axis=0, keepdims=True))
        cr = comp_col[i]
        val = iou * iou - cr * cr
        out_ref[...] = jnp.maximum(out_ref[...],
                                   jnp.max(val, axis=0, keepdims=True))

    @pl.when(i == j)
    def _diag():
        rr = lax.broadcasted_iota(jnp.int32, (_TILE, _TILE), 0)
        cc = lax.broadcasted_iota(jnp.int32, (_TILE, _TILE), 1)
        mask = rr < cc
        iou_m = jnp.where(mask, iou, 0.0)
        comp_new = jnp.maximum(comp_row[j],
                               jnp.max(iou_m, axis=0, keepdims=True))
        comp_col[j] = jnp.transpose(comp_new, (1, 0))
        cr = comp_col[j]
        val = jnp.where(mask, iou * iou - cr * cr, 0.0)
        m = jnp.maximum(out_ref[...], jnp.max(val, axis=0, keepdims=True))
        ns = s_c[...] * jnp.exp(m * (-1.0 / _SIGMA))
        out_ref[...] = jnp.where(ns >= _THRESH, ns, 0.0)


def _fused_sweep(x1s, y1s, x2s, y2s, ss):
    r = lambda a: a.reshape(_NP, 1)
    c = lambda a: a.reshape(1, _NP)
    row = pl.BlockSpec((_TILE, 1), lambda t, i_ref, j_ref: (i_ref[t], 0))
    col = pl.BlockSpec((1, _TILE), lambda t, i_ref, j_ref: (0, j_ref[t]))
    grid_spec = pltpu.PrefetchScalarGridSpec(
        num_scalar_prefetch=2,
        grid=(len(_TRI),),
        in_specs=[row] * 4 + [col] * 4 + [col],
        out_specs=col,
        scratch_shapes=[
            pltpu.VMEM((_NB, 1, _TILE), jnp.float32),
            pltpu.VMEM((_NB, _TILE, 1), jnp.float32),
        ],
    )
    return pl.pallas_call(
        _fused_body,
        grid_spec=grid_spec,
        out_shape=jax.ShapeDtypeStruct((1, _NP), jnp.float32),
    )(jnp.asarray(_I_ARR), jnp.asarray(_J_ARR),
      r(x1s), r(y1s), r(x2s), r(y2s), c(x1s), c(y1s), c(x2s), c(y2s), c(ss))


def kernel(boxes, scores):
    s_pad = jnp.concatenate(
        [scores, jnp.zeros((_NP - _N,), jnp.float32)])
    b_pad = jnp.concatenate(
        [boxes, jnp.zeros((_NP - _N, 4), jnp.float32)], axis=0)

    rank = _compute_rank(s_pad)

    vals = jnp.concatenate(
        [b_pad, s_pad.reshape(_NP, 1),
         jnp.zeros((_NP, 3), jnp.float32)], axis=1)
    sorted_vals = _sc_permute(
        rank.reshape(_ROWS, _COLS), vals.reshape(_ROWS, _COLS, 8))
    x1s, y1s, x2s, y2s = (sorted_vals[:, 0], sorted_vals[:, 1],
                          sorted_vals[:, 2], sorted_vals[:, 3])
    ss = sorted_vals[:, 4]

    out = _fused_sweep(x1s, y1s, x2s, y2s, ss)
    return out.reshape(_NP)[:_N]


# speedup vs baseline: 3.1683x; 1.0328x over previous
import functools

import numpy as np

import jax
import jax.numpy as jnp
from jax import lax
from jax.experimental import pallas as pl
from jax.experimental.pallas import tpu as pltpu
from jax.experimental.pallas import tpu_sc as plsc

_SIGMA = 0.5
_THRESH = 0.05
_N = 5000
_NP = 5120
_TILE = 1024
_NB = _NP // _TILE
_RT = 512
_ROWS = 64
_COLS = 80
_WROWS = 2


def _rank_body(s_col_ref, s_row_ref, rank_ref):
    i = pl.program_id(0)
    k_i = 2 * jax.lax.bitcast_convert_type(s_col_ref[...], jnp.int32)
    k_j = 2 * jax.lax.bitcast_convert_type(s_row_ref[...], jnp.int32)
    gi = i * _RT + lax.broadcasted_iota(jnp.int32, (_RT, _NP), 0)
    gj = lax.broadcasted_iota(jnp.int32, (_RT, _NP), 1)
    before = (k_j + (gj < gi).astype(jnp.int32)) > k_i
    rank_ref[...] = jnp.sum(before.astype(jnp.int32), axis=1, keepdims=True)


def _compute_rank(s_pad):
    return pl.pallas_call(
        _rank_body,
        grid=(_NP // _RT,),
        in_specs=[
            pl.BlockSpec((_RT, 1), lambda i: (i, 0)),
            pl.BlockSpec((1, _NP), lambda i: (0, 0)),
        ],
        out_specs=pl.BlockSpec((_RT, 1), lambda i: (i, 0)),
        out_shape=jax.ShapeDtypeStruct((_NP, 1), jnp.int32),
    )(s_pad.reshape(_NP, 1), s_pad.reshape(1, _NP))


def _sc_permute(rank2d, vals3d):
    mesh = plsc.VectorSubcoreMesh(core_axis_name="c", subcore_axis_name="s")

    @functools.partial(
        pl.kernel,
        mesh=mesh,
        out_type=jax.ShapeDtypeStruct((_NP, 8), jnp.float32),
        scratch_types=[
            pltpu.VMEM((_WROWS, _COLS), jnp.int32),
            pltpu.VMEM((_WROWS, _COLS, 8), jnp.float32),
            pltpu.SemaphoreType.DMA,
        ],
        compiler_params=pltpu.CompilerParams(use_tc_tiling_on_sc=False),
    )
    def permute(rank_h, vals_h, out_h, idx_v, val_v, sem):
        w = lax.axis_index("s") * 2 + lax.axis_index("c")
        base = w * _WROWS
        stg = [pltpu.async_copy(rank_h.at[pl.ds(base, _WROWS)], idx_v, sem),
               pltpu.async_copy(vals_h.at[pl.ds(base, _WROWS)], val_v, sem)]
        for cp in stg:
            cp.wait()
        cps = [pltpu.async_copy(val_v.at[r], out_h.at[idx_v.at[r]], sem)
               for r in range(_WROWS)]
        for cp in cps:
            cp.wait()

    return permute(rank2d, vals3d)


def _iou_tile(x1r, y1r, x2r, y2r, x1c, y1c, x2c, y2c):
    ix1 = jnp.maximum(x1r, x1c)
    iy1 = jnp.maximum(y1r, y1c)
    ix2 = jnp.minimum(x2r, x2c)
    iy2 = jnp.minimum(y2r, y2c)
    inter = jnp.maximum(ix2 - ix1, 0.0) * jnp.maximum(iy2 - iy1, 0.0)
    ar = (x2r - x1r) * (y2r - y1r)
    ac = (x2c - x1c) * (y2c - y1c)
    union = ar + ac - inter
    return inter * pl.reciprocal(union, approx=True)


def _tri_mask(i, j):
    rr = i * _TILE + lax.broadcasted_iota(jnp.int32, (_TILE, _TILE), 0)
    cc = j * _TILE + lax.broadcasted_iota(jnp.int32, (_TILE, _TILE), 1)
    return rr < cc


_TRI = [(i, j) for j in range(_NB) for i in range(j + 1)]
_I_ARR = np.array([t[0] for t in _TRI], np.int32)
_J_ARR = np.array([t[1] for t in _TRI], np.int32)


def _fused_body(i_ref, j_ref, x1r, y1r, x2r, y2r, x1c, y1c, x2c, y2c,
                s_c, out_ref, comp_row, comp_col):
    t = pl.program_id(0)
    i = i_ref[t]
    j = j_ref[t]

    @pl.when(t == 0)
    def _init_comp():
        comp_row[...] = jnp.zeros_like(comp_row[...])

    @pl.when(i == 0)
    def _init_m():
        out_ref[...] = jnp.zeros_like(out_ref[...])

    @pl.when(i < j)
    def _off_diag():
        iou = _iou_tile(x1r[...], y1r[...], x2r[...], y2r[...],
                        x1c[...], y1c[...], x2c[...], y2c[...])
        comp_row[j] = jnp.maximum(comp_row[j],
                                  jnp.max(iou, axis=0, keepdims=True))
        cr = comp_col[i]
        val = iou * iou - cr * cr
        out_ref[...] = jnp.maximum(out_ref[...],
                                   jnp.max(val, axis=0, keepdims=True))

    @pl.when(i == j)
    def _diag():
        iou = _iou_tile(x1r[...], y1r[...], x2r[...], y2r[...],
                        x1c[...], y1c[...], x2c[...], y2c[...])
        rr = lax.broadcasted_iota(jnp.int32, (_TILE, _TILE), 0)
        cc = lax.broadcasted_iota(jnp.int32, (_TILE, _TILE), 1)
        mask = rr < cc
        iou_m = jnp.where(mask, iou, 0.0)
        comp_new = jnp.maximum(comp_row[j],
                               jnp.max(iou_m, axis=0, keepdims=True))
        comp_col[j] = jnp.transpose(comp_new, (1, 0))
        cr = comp_col[j]
        val = jnp.where(mask, iou * iou - cr * cr, 0.0)
        m = jnp.maximum(out_ref[...], jnp.max(val, axis=0, keepdims=True))
        ns = s_c[...] * jnp.exp(m * (-1.0 / _SIGMA))
        out_ref[...] = jnp.where(ns >= _THRESH, ns, 0.0)


def _fused_sweep(x1s, y1s, x2s, y2s, ss):
    r = lambda a: a.reshape(_NP, 1)
    c = lambda a: a.reshape(1, _NP)
    row = pl.BlockSpec((_TILE, 1), lambda t, i_ref, j_ref: (i_ref[t], 0))
    col = pl.BlockSpec((1, _TILE), lambda t, i_ref, j_ref: (0, j_ref[t]))
    grid_spec = pltpu.PrefetchScalarGridSpec(
        num_scalar_prefetch=2,
        grid=(len(_TRI),),
        in_specs=[row] * 4 + [col] * 4 + [col],
        out_specs=col,
        scratch_shapes=[
            pltpu.VMEM((_NB, 1, _TILE), jnp.float32),
            pltpu.VMEM((_NB, _TILE, 1), jnp.float32),
        ],
    )
    return pl.pallas_call(
        _fused_body,
        grid_spec=grid_spec,
        out_shape=jax.ShapeDtypeStruct((1, _NP), jnp.float32),
    )(jnp.asarray(_I_ARR), jnp.asarray(_J_ARR),
      r(x1s), r(y1s), r(x2s), r(y2s), c(x1s), c(y1s), c(x2s), c(y2s), c(ss))


def kernel(boxes, scores):
    s_pad = jnp.concatenate(
        [scores, jnp.zeros((_NP - _N,), jnp.float32)])
    b_pad = jnp.concatenate(
        [boxes, jnp.zeros((_NP - _N, 4), jnp.float32)], axis=0)

    rank = _compute_rank(s_pad)

    vals = jnp.concatenate(
        [b_pad, s_pad.reshape(_NP, 1),
         jnp.zeros((_NP, 3), jnp.float32)], axis=1)
    sorted_vals = _sc_permute(
        rank.reshape(_ROWS, _COLS), vals.reshape(_ROWS, _COLS, 8))
    x1s, y1s, x2s, y2s = (sorted_vals[:, 0], sorted_vals[:, 1],
                          sorted_vals[:, 2], sorted_vals[:, 3])
    ss = sorted_vals[:, 4]

    out = _fused_sweep(x1s, y1s, x2s, y2s, ss)
    return out.reshape(_NP)[:_N]


# speedup vs baseline: 3.4456x; 1.0875x over previous
import functools

import numpy as np

import jax
import jax.numpy as jnp
from jax import lax
from jax.experimental import pallas as pl
from jax.experimental.pallas import tpu as pltpu
from jax.experimental.pallas import tpu_sc as plsc

_SIGMA = 0.5
_THRESH = 0.05
_N = 5000
_NP = 5120
_TILE = 1024
_NB = _NP // _TILE
_RT = 512
_ROWS = 64
_COLS = 80
_WROWS = 2


def _rank_body(s_col_ref, s_row_ref, rank_ref):
    i = pl.program_id(0)
    k_i = 2 * jax.lax.bitcast_convert_type(s_col_ref[...], jnp.int32)
    k_j = 2 * jax.lax.bitcast_convert_type(s_row_ref[...], jnp.int32)
    gi = i * _RT + lax.broadcasted_iota(jnp.int32, (_RT, _NP), 0)
    gj = lax.broadcasted_iota(jnp.int32, (_RT, _NP), 1)
    before = (k_j + (gj < gi).astype(jnp.int32)) > k_i
    rank_ref[...] = jnp.sum(before.astype(jnp.int32), axis=1, keepdims=True)


def _compute_rank(s_pad):
    return pl.pallas_call(
        _rank_body,
        grid=(_NP // _RT,),
        in_specs=[
            pl.BlockSpec((_RT, 1), lambda i: (i, 0)),
            pl.BlockSpec((1, _NP), lambda i: (0, 0)),
        ],
        out_specs=pl.BlockSpec((_RT, 1), lambda i: (i, 0)),
        out_shape=jax.ShapeDtypeStruct((_NP, 1), jnp.int32),
    )(s_pad.reshape(_NP, 1), s_pad.reshape(1, _NP))


def _sc_permute(rank2d, vals3d):
    mesh = plsc.VectorSubcoreMesh(core_axis_name="c", subcore_axis_name="s")

    @functools.partial(
        pl.kernel,
        mesh=mesh,
        out_type=jax.ShapeDtypeStruct((_NP, 8), jnp.float32),
        scratch_types=[
            pltpu.VMEM((_WROWS, _COLS), jnp.int32),
            pltpu.VMEM((_WROWS, _COLS, 8), jnp.float32),
            pltpu.SemaphoreType.DMA,
        ],
        compiler_params=pltpu.CompilerParams(use_tc_tiling_on_sc=False),
    )
    def permute(rank_h, vals_h, out_h, idx_v, val_v, sem):
        w = lax.axis_index("s") * 2 + lax.axis_index("c")
        base = w * _WROWS
        stg = [pltpu.async_copy(rank_h.at[pl.ds(base, _WROWS)], idx_v, sem),
               pltpu.async_copy(vals_h.at[pl.ds(base, _WROWS)], val_v, sem)]
        for cp in stg:
            cp.wait()
        cps = [pltpu.async_copy(val_v.at[r], out_h.at[idx_v.at[r]], sem)
               for r in range(_WROWS)]
        for cp in cps:
            cp.wait()

    return permute(rank2d, vals3d)


def _iou_tile(x1r, y1r, x2r, y2r, x1c, y1c, x2c, y2c):
    ix1 = jnp.maximum(x1r, x1c)
    iy1 = jnp.maximum(y1r, y1c)
    ix2 = jnp.minimum(x2r, x2c)
    iy2 = jnp.minimum(y2r, y2c)
    inter = jnp.maximum(ix2 - ix1, 0.0) * jnp.maximum(iy2 - iy1, 0.0)
    ar = (x2r - x1r) * (y2r - y1r)
    ac = (x2c - x1c) * (y2c - y1c)
    union = ar + ac - inter
    return inter * pl.reciprocal(union, approx=True)


def _tri_mask(i, j):
    rr = i * _TILE + lax.broadcasted_iota(jnp.int32, (_TILE, _TILE), 0)
    cc = j * _TILE + lax.broadcasted_iota(jnp.int32, (_TILE, _TILE), 1)
    return rr < cc


_TRI = [(i, j) for j in range(_NB) for i in range(j + 1)]
_I_ARR = np.array([t[0] for t in _TRI], np.int32)
_J_ARR = np.array([t[1] for t in _TRI], np.int32)


def _fused_body(i_ref, j_ref, rows, cols, out_ref, comp_row, comp_col):
    t = pl.program_id(0)
    i = i_ref[t]
    j = j_ref[t]

    def iou_tile():
        rb = rows[...]
        cb = cols[...]
        return _iou_tile(rb[:, 0:1], rb[:, 1:2], rb[:, 2:3], rb[:, 3:4],
                         cb[0:1, :], cb[1:2, :], cb[2:3, :], cb[3:4, :])

    @pl.when(t == 0)
    def _init_comp():
        comp_row[...] = jnp.zeros_like(comp_row[...])

    @pl.when(i == 0)
    def _init_m():
        out_ref[...] = jnp.zeros_like(out_ref[...])

    @pl.when(i < j)
    def _off_diag():
        iou = iou_tile()
        comp_row[j] = jnp.maximum(comp_row[j],
                                  jnp.max(iou, axis=0, keepdims=True))
        cr = comp_col[i]
        val = iou * iou - cr * cr
        out_ref[...] = jnp.maximum(out_ref[...],
                                   jnp.max(val, axis=0, keepdims=True))

    @pl.when(i == j)
    def _diag():
        iou = iou_tile()
        rr = lax.broadcasted_iota(jnp.int32, (_TILE, _TILE), 0)
        cc = lax.broadcasted_iota(jnp.int32, (_TILE, _TILE), 1)
        mask = rr < cc
        iou_m = jnp.where(mask, iou, 0.0)
        comp_new = jnp.maximum(comp_row[j],
                               jnp.max(iou_m, axis=0, keepdims=True))
        comp_col[j] = jnp.transpose(comp_new, (1, 0))
        cr = comp_col[j]
        val = jnp.where(mask, iou * iou - cr * cr, 0.0)
        m = jnp.maximum(out_ref[...], jnp.max(val, axis=0, keepdims=True))
        ns = cols[4:5, :] * jnp.exp(m * (-1.0 / _SIGMA))
        out_ref[...] = jnp.where(ns >= _THRESH, ns, 0.0)


def _fused_sweep(sorted_vals, sorted_vals_t):
    row = pl.BlockSpec((_TILE, 8), lambda t, i_ref, j_ref: (i_ref[t], 0))
    col = pl.BlockSpec((8, _TILE), lambda t, i_ref, j_ref: (0, j_ref[t]))
    grid_spec = pltpu.PrefetchScalarGridSpec(
        num_scalar_prefetch=2,
        grid=(len(_TRI),),
        in_specs=[row, col],
        out_specs=pl.BlockSpec((1, _TILE), lambda t, i_ref, j_ref:
                               (0, j_ref[t])),
        scratch_shapes=[
            pltpu.VMEM((_NB, 1, _TILE), jnp.float32),
            pltpu.VMEM((_NB, _TILE, 1), jnp.float32),
        ],
    )
    return pl.pallas_call(
        _fused_body,
        grid_spec=grid_spec,
        out_shape=jax.ShapeDtypeStruct((1, _NP), jnp.float32),
    )(jnp.asarray(_I_ARR), jnp.asarray(_J_ARR), sorted_vals, sorted_vals_t)


def kernel(boxes, scores):
    s_pad = jnp.concatenate(
        [scores, jnp.zeros((_NP - _N,), jnp.float32)])
    b_pad = jnp.concatenate(
        [boxes, jnp.zeros((_NP - _N, 4), jnp.float32)], axis=0)

    rank = _compute_rank(s_pad)

    vals = jnp.concatenate(
        [b_pad, s_pad.reshape(_NP, 1),
         jnp.zeros((_NP, 3), jnp.float32)], axis=1)
    sorted_vals = _sc_permute(
        rank.reshape(_ROWS, _COLS), vals.reshape(_ROWS, _COLS, 8))

    out = _fused_sweep(sorted_vals, sorted_vals.T)
    return out.reshape(_NP)[:_N]


# speedup vs baseline: 3.5218x; 1.0221x over previous
import functools

import numpy as np

import jax
import jax.numpy as jnp
from jax import lax
from jax.experimental import pallas as pl
from jax.experimental.pallas import tpu as pltpu
from jax.experimental.pallas import tpu_sc as plsc

_SIGMA = 0.5
_THRESH = 0.05
_N = 5000
_NP = 5120
_TILE = 1024
_NB = _NP // _TILE
_RT = 512
_ROWS = 64
_COLS = 80
_WROWS = 2


def _rank_body(s_col_ref, s_row_ref, rank_ref):
    i = pl.program_id(0)
    k_i = 2 * jax.lax.bitcast_convert_type(s_col_ref[...], jnp.int32)
    k_j = 2 * jax.lax.bitcast_convert_type(s_row_ref[...], jnp.int32)
    gi = i * _RT + lax.broadcasted_iota(jnp.int32, (_RT, _NP), 0)
    gj = lax.broadcasted_iota(jnp.int32, (_RT, _NP), 1)
    before = (k_j + (gj < gi).astype(jnp.int32)) > k_i
    rank_ref[...] = jnp.sum(before.astype(jnp.int32), axis=1, keepdims=True)


def _compute_rank(s_pad):
    return pl.pallas_call(
        _rank_body,
        grid=(_NP // _RT,),
        in_specs=[
            pl.BlockSpec((_RT, 1), lambda i: (i, 0)),
            pl.BlockSpec((1, _NP), lambda i: (0, 0)),
        ],
        out_specs=pl.BlockSpec((_RT, 1), lambda i: (i, 0)),
        out_shape=jax.ShapeDtypeStruct((_NP, 1), jnp.int32),
        compiler_params=pltpu.CompilerParams(
            allow_input_fusion=[True, True]),
    )(s_pad.reshape(_NP, 1), s_pad.reshape(1, _NP))


def _sc_permute(rank2d, vals3d):
    mesh = plsc.VectorSubcoreMesh(core_axis_name="c", subcore_axis_name="s")

    @functools.partial(
        pl.kernel,
        mesh=mesh,
        out_type=jax.ShapeDtypeStruct((_NP, 8), jnp.float32),
        scratch_types=[
            pltpu.VMEM((_WROWS, _COLS), jnp.int32),
            pltpu.VMEM((_WROWS, _COLS, 8), jnp.float32),
            pltpu.SemaphoreType.DMA,
        ],
        compiler_params=pltpu.CompilerParams(use_tc_tiling_on_sc=False),
    )
    def permute(rank_h, vals_h, out_h, idx_v, val_v, sem):
        w = lax.axis_index("s") * 2 + lax.axis_index("c")
        base = w * _WROWS
        stg = [pltpu.async_copy(rank_h.at[pl.ds(base, _WROWS)], idx_v, sem),
               pltpu.async_copy(vals_h.at[pl.ds(base, _WROWS)], val_v, sem)]
        for cp in stg:
            cp.wait()
        cps = [pltpu.async_copy(val_v.at[r], out_h.at[idx_v.at[r]], sem)
               for r in range(_WROWS)]
        for cp in cps:
            cp.wait()

    return permute(rank2d, vals3d)


def _iou_tile(x1r, y1r, x2r, y2r, x1c, y1c, x2c, y2c):
    ix1 = jnp.maximum(x1r, x1c)
    iy1 = jnp.maximum(y1r, y1c)
    ix2 = jnp.minimum(x2r, x2c)
    iy2 = jnp.minimum(y2r, y2c)
    inter = jnp.maximum(ix2 - ix1, 0.0) * jnp.maximum(iy2 - iy1, 0.0)
    ar = (x2r - x1r) * (y2r - y1r)
    ac = (x2c - x1c) * (y2c - y1c)
    union = ar + ac - inter
    return inter * pl.reciprocal(union, approx=True)


def _tri_mask(i, j):
    rr = i * _TILE + lax.broadcasted_iota(jnp.int32, (_TILE, _TILE), 0)
    cc = j * _TILE + lax.broadcasted_iota(jnp.int32, (_TILE, _TILE), 1)
    return rr < cc


_TRI = [(i, j) for j in range(_NB) for i in range(j + 1)]
_I_ARR = np.array([t[0] for t in _TRI], np.int32)
_J_ARR = np.array([t[1] for t in _TRI], np.int32)


def _fused_body(i_ref, j_ref, rows, cols, out_ref, comp_row, comp_col):
    t = pl.program_id(0)
    i = i_ref[t]
    j = j_ref[t]

    def iou_tile():
        rb = rows[...]
        cb = cols[...]
        return _iou_tile(rb[:, 0:1], rb[:, 1:2], rb[:, 2:3], rb[:, 3:4],
                         cb[0:1, :], cb[1:2, :], cb[2:3, :], cb[3:4, :])

    @pl.when(t == 0)
    def _init_comp():
        comp_row[...] = jnp.zeros_like(comp_row[...])

    @pl.when(i == 0)
    def _init_m():
        out_ref[...] = jnp.zeros_like(out_ref[...])

    @pl.when(i < j)
    def _off_diag():
        iou = iou_tile()
        comp_row[j] = jnp.maximum(comp_row[j],
                                  jnp.max(iou, axis=0, keepdims=True))
        cr = comp_col[i]
        val = iou * iou - cr * cr
        out_ref[...] = jnp.maximum(out_ref[...],
                                   jnp.max(val, axis=0, keepdims=True))

    @pl.when(i == j)
    def _diag():
        iou = iou_tile()
        rr = lax.broadcasted_iota(jnp.int32, (_TILE, _TILE), 0)
        cc = lax.broadcasted_iota(jnp.int32, (_TILE, _TILE), 1)
        mask = rr < cc
        iou_m = jnp.where(mask, iou, 0.0)
        comp_new = jnp.maximum(comp_row[j],
                               jnp.max(iou_m, axis=0, keepdims=True))
        comp_col[j] = jnp.transpose(comp_new, (1, 0))
        cr = comp_col[j]
        val = jnp.where(mask, iou * iou - cr * cr, 0.0)
        m = jnp.maximum(out_ref[...], jnp.max(val, axis=0, keepdims=True))
        ns = cols[4:5, :] * jnp.exp(m * (-1.0 / _SIGMA))
        out_ref[...] = jnp.where(ns >= _THRESH, ns, 0.0)


def _fused_sweep(sorted_vals, sorted_vals_t):
    row = pl.BlockSpec((_TILE, 8), lambda t, i_ref, j_ref: (i_ref[t], 0))
    col = pl.BlockSpec((8, _TILE), lambda t, i_ref, j_ref: (0, j_ref[t]))
    grid_spec = pltpu.PrefetchScalarGridSpec(
        num_scalar_prefetch=2,
        grid=(len(_TRI),),
        in_specs=[row, col],
        out_specs=pl.BlockSpec((1, _TILE), lambda t, i_ref, j_ref:
                               (0, j_ref[t])),
        scratch_shapes=[
            pltpu.VMEM((_NB, 1, _TILE), jnp.float32),
            pltpu.VMEM((_NB, _TILE, 1), jnp.float32),
        ],
    )
    return pl.pallas_call(
        _fused_body,
        grid_spec=grid_spec,
        out_shape=jax.ShapeDtypeStruct((1, _NP), jnp.float32),
        compiler_params=pltpu.CompilerParams(
            allow_input_fusion=[False, False, True, True]),
    )(jnp.asarray(_I_ARR), jnp.asarray(_J_ARR), sorted_vals, sorted_vals_t)


def kernel(boxes, scores):
    s_pad = jnp.concatenate(
        [scores, jnp.zeros((_NP - _N,), jnp.float32)])
    b_pad = jnp.concatenate(
        [boxes, jnp.zeros((_NP - _N, 4), jnp.float32)], axis=0)

    rank = _compute_rank(s_pad)

    vals = jnp.concatenate(
        [b_pad, s_pad.reshape(_NP, 1),
         jnp.zeros((_NP, 3), jnp.float32)], axis=1)
    sorted_vals = _sc_permute(
        rank.reshape(_ROWS, _COLS), vals.reshape(_ROWS, _COLS, 8))

    out = _fused_sweep(sorted_vals, sorted_vals.T)
    return out.reshape(_NP)[:_N]


# speedup vs baseline: 3.7714x; 1.0709x over previous
import functools

import numpy as np

import jax
import jax.numpy as jnp
from jax import lax
from jax.experimental import pallas as pl
from jax.experimental.pallas import tpu as pltpu
from jax.experimental.pallas import tpu_sc as plsc

_SIGMA = 0.5
_THRESH = 0.05
_N = 5000
_NP = 5120
_TILE = 1024
_NB = _NP // _TILE
_RT = 512
_ROWS = 64
_COLS = 80
_WROWS = 2


def _rank_body(s_col_ref, s_row_ref, b_ref, rank_ref, vals_ref):
    i = pl.program_id(0)
    k_i = 2 * jax.lax.bitcast_convert_type(s_col_ref[...], jnp.int32)
    k_j = 2 * jax.lax.bitcast_convert_type(s_row_ref[...], jnp.int32)
    gi = i * _RT + lax.broadcasted_iota(jnp.int32, (_RT, _NP), 0)
    gj = lax.broadcasted_iota(jnp.int32, (_RT, _NP), 1)
    before = (k_j + (gj < gi).astype(jnp.int32)) > k_i
    rank_ref[...] = jnp.sum(before.astype(jnp.int32), axis=1, keepdims=True)
    vals_ref[...] = jnp.concatenate(
        [b_ref[...], s_col_ref[...], jnp.zeros((_RT, 3), jnp.float32)],
        axis=1)


def _compute_rank(s_pad, b_pad):
    return pl.pallas_call(
        _rank_body,
        grid=(_NP // _RT,),
        in_specs=[
            pl.BlockSpec((_RT, 1), lambda i: (i, 0)),
            pl.BlockSpec((1, _NP), lambda i: (0, 0)),
            pl.BlockSpec((_RT, 4), lambda i: (i, 0)),
        ],
        out_specs=[
            pl.BlockSpec((_RT, 1), lambda i: (i, 0)),
            pl.BlockSpec((_RT, 8), lambda i: (i, 0)),
        ],
        out_shape=[
            jax.ShapeDtypeStruct((_NP, 1), jnp.int32),
            jax.ShapeDtypeStruct((_NP, 8), jnp.float32),
        ],
        compiler_params=pltpu.CompilerParams(
            allow_input_fusion=[True, True, True]),
    )(s_pad.reshape(_NP, 1), s_pad.reshape(1, _NP), b_pad)


def _sc_permute(rank2d, vals3d):
    mesh = plsc.VectorSubcoreMesh(core_axis_name="c", subcore_axis_name="s")

    @functools.partial(
        pl.kernel,
        mesh=mesh,
        out_type=jax.ShapeDtypeStruct((_NP, 8), jnp.float32),
        scratch_types=[
            pltpu.VMEM((_WROWS, _COLS), jnp.int32),
            pltpu.VMEM((_WROWS, _COLS, 8), jnp.float32),
            pltpu.SemaphoreType.DMA,
        ],
        compiler_params=pltpu.CompilerParams(use_tc_tiling_on_sc=False),
    )
    def permute(rank_h, vals_h, out_h, idx_v, val_v, sem):
        w = lax.axis_index("s") * 2 + lax.axis_index("c")
        base = w * _WROWS
        stg = [pltpu.async_copy(rank_h.at[pl.ds(base, _WROWS)], idx_v, sem),
               pltpu.async_copy(vals_h.at[pl.ds(base, _WROWS)], val_v, sem)]
        for cp in stg:
            cp.wait()
        cps = [pltpu.async_copy(val_v.at[r], out_h.at[idx_v.at[r]], sem)
               for r in range(_WROWS)]
        for cp in cps:
            cp.wait()

    return permute(rank2d, vals3d)


def _iou_tile(x1r, y1r, x2r, y2r, x1c, y1c, x2c, y2c):
    ix1 = jnp.maximum(x1r, x1c)
    iy1 = jnp.maximum(y1r, y1c)
    ix2 = jnp.minimum(x2r, x2c)
    iy2 = jnp.minimum(y2r, y2c)
    inter = jnp.maximum(ix2 - ix1, 0.0) * jnp.maximum(iy2 - iy1, 0.0)
    ar = (x2r - x1r) * (y2r - y1r)
    ac = (x2c - x1c) * (y2c - y1c)
    union = ar + ac - inter
    return inter * pl.reciprocal(union, approx=True)


def _tri_mask(i, j):
    rr = i * _TILE + lax.broadcasted_iota(jnp.int32, (_TILE, _TILE), 0)
    cc = j * _TILE + lax.broadcasted_iota(jnp.int32, (_TILE, _TILE), 1)
    return rr < cc


_TRI = [(i, j) for j in range(_NB) for i in range(j + 1)]
_I_ARR = np.array([t[0] for t in _TRI], np.int32)
_J_ARR = np.array([t[1] for t in _TRI], np.int32)


def _fused_body(i_ref, j_ref, rows, cols, out_ref, comp_row, comp_col):
    t = pl.program_id(0)
    i = i_ref[t]
    j = j_ref[t]

    def iou_tile():
        rb = rows[...]
        cb = cols[...]
        return _iou_tile(rb[:, 0:1], rb[:, 1:2], rb[:, 2:3], rb[:, 3:4],
                         cb[0:1, :], cb[1:2, :], cb[2:3, :], cb[3:4, :])

    @pl.when(t == 0)
    def _init_comp():
        comp_row[...] = jnp.zeros_like(comp_row[...])

    @pl.when(i == 0)
    def _init_m():
        out_ref[...] = jnp.zeros_like(out_ref[...])

    @pl.when(i < j)
    def _off_diag():
        iou = iou_tile()
        comp_row[j] = jnp.maximum(comp_row[j],
                                  jnp.max(iou, axis=0, keepdims=True))
        cr = comp_col[i]
        val = iou * iou - cr * cr
        out_ref[...] = jnp.maximum(out_ref[...],
                                   jnp.max(val, axis=0, keepdims=True))

    @pl.when(i == j)
    def _diag():
        iou = iou_tile()
        rr = lax.broadcasted_iota(jnp.int32, (_TILE, _TILE), 0)
        cc = lax.broadcasted_iota(jnp.int32, (_TILE, _TILE), 1)
        mask = rr < cc
        iou_m = jnp.where(mask, iou, 0.0)
        comp_new = jnp.maximum(comp_row[j],
                               jnp.max(iou_m, axis=0, keepdims=True))
        comp_col[j] = jnp.transpose(comp_new, (1, 0))
        cr = comp_col[j]
        val = jnp.where(mask, iou * iou - cr * cr, 0.0)
        m = jnp.maximum(out_ref[...], jnp.max(val, axis=0, keepdims=True))
        ns = cols[4:5, :] * jnp.exp(m * (-1.0 / _SIGMA))
        out_ref[...] = jnp.where(ns >= _THRESH, ns, 0.0)


def _fused_sweep(sorted_vals, sorted_vals_t):
    row = pl.BlockSpec((_TILE, 8), lambda t, i_ref, j_ref: (i_ref[t], 0))
    col = pl.BlockSpec((8, _TILE), lambda t, i_ref, j_ref: (0, j_ref[t]))
    grid_spec = pltpu.PrefetchScalarGridSpec(
        num_scalar_prefetch=2,
        grid=(len(_TRI),),
        in_specs=[row, col],
        out_specs=pl.BlockSpec((1, _TILE), lambda t, i_ref, j_ref:
                               (0, j_ref[t])),
        scratch_shapes=[
            pltpu.VMEM((_NB, 1, _TILE), jnp.float32),
            pltpu.VMEM((_NB, _TILE, 1), jnp.float32),
        ],
    )
    return pl.pallas_call(
        _fused_body,
        grid_spec=grid_spec,
        out_shape=jax.ShapeDtypeStruct((1, _NP), jnp.float32),
        compiler_params=pltpu.CompilerParams(
            allow_input_fusion=[False, False, True, True]),
    )(jnp.asarray(_I_ARR), jnp.asarray(_J_ARR), sorted_vals, sorted_vals_t)


def kernel(boxes, scores):
    s_pad = jnp.concatenate(
        [scores, jnp.zeros((_NP - _N,), jnp.float32)])
    b_pad = jnp.concatenate(
        [boxes, jnp.zeros((_NP - _N, 4), jnp.float32)], axis=0)

    rank, vals = _compute_rank(s_pad, b_pad)

    sorted_vals = _sc_permute(
        rank.reshape(_ROWS, _COLS), vals.reshape(_ROWS, _COLS, 8))

    out = _fused_sweep(sorted_vals, sorted_vals.T)
    return out.reshape(_NP)[:_N]


# speedup vs baseline: 3.7866x; 1.0040x over previous
import functools

import numpy as np

import jax
import jax.numpy as jnp
from jax import lax
from jax.experimental import pallas as pl
from jax.experimental.pallas import tpu as pltpu
from jax.experimental.pallas import tpu_sc as plsc

_SIGMA = 0.5
_THRESH = 0.05
_N = 5000
_NP = 5120
_TILE = 1024
_NB = _NP // _TILE
_RT = 1024
_ROWS = 64
_COLS = 80
_WROWS = 2


def _rank_body(s_col_ref, s_row_ref, b_ref, rank_ref, vals_ref):
    i = pl.program_id(0)
    k_i = 2 * jax.lax.bitcast_convert_type(s_col_ref[...], jnp.int32)
    k_j = 2 * jax.lax.bitcast_convert_type(s_row_ref[...], jnp.int32)
    gi = i * _RT + lax.broadcasted_iota(jnp.int32, (_RT, _NP), 0)
    gj = lax.broadcasted_iota(jnp.int32, (_RT, _NP), 1)
    before = (k_j + (gj < gi).astype(jnp.int32)) > k_i
    rank_ref[...] = jnp.sum(before.astype(jnp.int32), axis=1, keepdims=True)
    vals_ref[...] = jnp.concatenate(
        [b_ref[...], s_col_ref[...], jnp.zeros((_RT, 3), jnp.float32)],
        axis=1)


def _compute_rank(s_pad, b_pad):
    return pl.pallas_call(
        _rank_body,
        grid=(_NP // _RT,),
        in_specs=[
            pl.BlockSpec((_RT, 1), lambda i: (i, 0)),
            pl.BlockSpec((1, _NP), lambda i: (0, 0)),
            pl.BlockSpec((_RT, 4), lambda i: (i, 0)),
        ],
        out_specs=[
            pl.BlockSpec((_RT, 1), lambda i: (i, 0)),
            pl.BlockSpec((_RT, 8), lambda i: (i, 0)),
        ],
        out_shape=[
            jax.ShapeDtypeStruct((_NP, 1), jnp.int32),
            jax.ShapeDtypeStruct((_NP, 8), jnp.float32),
        ],
        compiler_params=pltpu.CompilerParams(
            allow_input_fusion=[True, True, True]),
    )(s_pad.reshape(_NP, 1), s_pad.reshape(1, _NP), b_pad)


def _sc_permute(rank2d, vals3d):
    mesh = plsc.VectorSubcoreMesh(core_axis_name="c", subcore_axis_name="s")

    @functools.partial(
        pl.kernel,
        mesh=mesh,
        out_type=jax.ShapeDtypeStruct((_NP, 8), jnp.float32),
        scratch_types=[
            pltpu.VMEM((_WROWS, _COLS), jnp.int32),
            pltpu.VMEM((_WROWS, _COLS, 8), jnp.float32),
            pltpu.SemaphoreType.DMA,
        ],
        compiler_params=pltpu.CompilerParams(use_tc_tiling_on_sc=False),
    )
    def permute(rank_h, vals_h, out_h, idx_v, val_v, sem):
        w = lax.axis_index("s") * 2 + lax.axis_index("c")
        base = w * _WROWS
        stg = [pltpu.async_copy(rank_h.at[pl.ds(base, _WROWS)], idx_v, sem),
               pltpu.async_copy(vals_h.at[pl.ds(base, _WROWS)], val_v, sem)]
        for cp in stg:
            cp.wait()
        cps = [pltpu.async_copy(val_v.at[r], out_h.at[idx_v.at[r]], sem)
               for r in range(_WROWS)]
        for cp in cps:
            cp.wait()

    return permute(rank2d, vals3d)


def _iou_tile(x1r, y1r, x2r, y2r, x1c, y1c, x2c, y2c):
    ix1 = jnp.maximum(x1r, x1c)
    iy1 = jnp.maximum(y1r, y1c)
    ix2 = jnp.minimum(x2r, x2c)
    iy2 = jnp.minimum(y2r, y2c)
    inter = jnp.maximum(ix2 - ix1, 0.0) * jnp.maximum(iy2 - iy1, 0.0)
    ar = (x2r - x1r) * (y2r - y1r)
    ac = (x2c - x1c) * (y2c - y1c)
    union = ar + ac - inter
    return inter * pl.reciprocal(union, approx=True)


def _tri_mask(i, j):
    rr = i * _TILE + lax.broadcasted_iota(jnp.int32, (_TILE, _TILE), 0)
    cc = j * _TILE + lax.broadcasted_iota(jnp.int32, (_TILE, _TILE), 1)
    return rr < cc


_TRI = [(i, j) for j in range(_NB) for i in range(j + 1)]
_I_ARR = np.array([t[0] for t in _TRI], np.int32)
_J_ARR = np.array([t[1] for t in _TRI], np.int32)


def _fused_body(i_ref, j_ref, rows, cols, out_ref, comp_row, comp_col):
    t = pl.program_id(0)
    i = i_ref[t]
    j = j_ref[t]

    def iou_tile():
        rb = rows[...]
        cb = cols[...]
        return _iou_tile(rb[:, 0:1], rb[:, 1:2], rb[:, 2:3], rb[:, 3:4],
                         cb[0:1, :], cb[1:2, :], cb[2:3, :], cb[3:4, :])

    @pl.when(t == 0)
    def _init_comp():
        comp_row[...] = jnp.zeros_like(comp_row[...])

    @pl.when(i == 0)
    def _init_m():
        out_ref[...] = jnp.zeros_like(out_ref[...])

    @pl.when(i < j)
    def _off_diag():
        iou = iou_tile()
        comp_row[j] = jnp.maximum(comp_row[j],
                                  jnp.max(iou, axis=0, keepdims=True))
        cr = comp_col[i]
        val = iou * iou - cr * cr
        out_ref[...] = jnp.maximum(out_ref[...],
                                   jnp.max(val, axis=0, keepdims=True))

    @pl.when(i == j)
    def _diag():
        iou = iou_tile()
        rr = lax.broadcasted_iota(jnp.int32, (_TILE, _TILE), 0)
        cc = lax.broadcasted_iota(jnp.int32, (_TILE, _TILE), 1)
        mask = rr < cc
        iou_m = jnp.where(mask, iou, 0.0)
        comp_new = jnp.maximum(comp_row[j],
                               jnp.max(iou_m, axis=0, keepdims=True))
        comp_col[j] = jnp.transpose(comp_new, (1, 0))
        cr = comp_col[j]
        val = jnp.where(mask, iou * iou - cr * cr, 0.0)
        m = jnp.maximum(out_ref[...], jnp.max(val, axis=0, keepdims=True))
        ns = cols[4:5, :] * jnp.exp(m * (-1.0 / _SIGMA))
        out_ref[...] = jnp.where(ns >= _THRESH, ns, 0.0)


def _fused_sweep(sorted_vals, sorted_vals_t):
    row = pl.BlockSpec((_TILE, 8), lambda t, i_ref, j_ref: (i_ref[t], 0))
    col = pl.BlockSpec((8, _TILE), lambda t, i_ref, j_ref: (0, j_ref[t]))
    grid_spec = pltpu.PrefetchScalarGridSpec(
        num_scalar_prefetch=2,
        grid=(len(_TRI),),
        in_specs=[row, col],
        out_specs=pl.BlockSpec((1, _TILE), lambda t, i_ref, j_ref:
                               (0, j_ref[t])),
        scratch_shapes=[
            pltpu.VMEM((_NB, 1, _TILE), jnp.float32),
            pltpu.VMEM((_NB, _TILE, 1), jnp.float32),
        ],
    )
    return pl.pallas_call(
        _fused_body,
        grid_spec=grid_spec,
        out_shape=jax.ShapeDtypeStruct((1, _NP), jnp.float32),
        compiler_params=pltpu.CompilerParams(
            allow_input_fusion=[False, False, True, True]),
    )(jnp.asarray(_I_ARR), jnp.asarray(_J_ARR), sorted_vals, sorted_vals_t)


def kernel(boxes, scores):
    s_pad = jnp.concatenate(
        [scores, jnp.zeros((_NP - _N,), jnp.float32)])
    b_pad = jnp.concatenate(
        [boxes, jnp.zeros((_NP - _N, 4), jnp.float32)], axis=0)

    rank, vals = _compute_rank(s_pad, b_pad)

    sorted_vals = _sc_permute(
        rank.reshape(_ROWS, _COLS), vals.reshape(_ROWS, _COLS, 8))

    out = _fused_sweep(sorted_vals, sorted_vals.T)
    return out.reshape(_NP)[:_N]
